# baseline (speedup 1.0000x reference)
import functools

import jax
import jax.numpy as jnp
import numpy as np
from jax import lax
from jax.experimental import pallas as pl
from jax.experimental.pallas import tpu as pltpu

_TT = 512
_CH = 512
_NF = 10000
_NB = 2000
_FPAD = 10240
_BPAD = 2048
_RBF = 8


def _pair_acc(acc, tx, ty, sx, sy, sf, w, rsq, inv_s):
    dx = sx - tx
    dy = sy - ty
    d2 = dx * dx + dy * dy
    mask = (d2 < rsq).astype(jnp.float32)
    r = jnp.sqrt(d2) * inv_s
    u = 2.0 * r - 1.0
    v = jnp.arctan2(dy, dx) * jnp.float32(1.0 / np.pi)
    h_inv = jnp.float32((_RBF - 1) / 2.0)
    centers = np.linspace(-1.0, 1.0, _RBF)
    bv = [jnp.maximum(0.0, 1.0 - jnp.abs(v - jnp.float32(c)) * h_inv)
          for c in centers]
    t = jnp.zeros_like(d2)
    for n in range(_RBF):
        bu = jnp.maximum(0.0, 1.0 - jnp.abs(u - jnp.float32(centers[n])) * h_inv)
        row = jnp.zeros_like(d2)
        for m in range(_RBF):
            row = row + w[n][m] * bv[m]
        t = t + bu * row
    return acc + t * mask * sf


def _dense_kernel(sup_ref, wf_ref, wb_ref, tx_ref, ty_ref,
                  fsx_ref, fsy_ref, fsf_ref, bsx_ref, bsy_ref, bsf_ref,
                  out_ref):
    tx = tx_ref[:, :]
    ty = ty_ref[:, :]
    s = sup_ref[0]
    rsq = s * s
    inv_s = 1.0 / s
    wf = [[wf_ref[n, m] for m in range(_RBF)] for n in range(_RBF)]
    wb = [[wb_ref[n, m] for m in range(_RBF)] for n in range(_RBF)]

    def floop(k, acc):
        sx = fsx_ref[:, pl.ds(k * _CH, _CH)]
        sy = fsy_ref[:, pl.ds(k * _CH, _CH)]
        sf = fsf_ref[:, pl.ds(k * _CH, _CH)]
        return _pair_acc(acc, tx, ty, sx, sy, sf, wf, rsq, inv_s)

    def bloop(k, acc):
        sx = bsx_ref[:, pl.ds(k * _CH, _CH)]
        sy = bsy_ref[:, pl.ds(k * _CH, _CH)]
        sf = bsf_ref[:, pl.ds(k * _CH, _CH)]
        return _pair_acc(acc, tx, ty, sx, sy, sf, wb, rsq, inv_s)

    acc = jnp.zeros((_TT, _CH), jnp.float32)
    acc = lax.fori_loop(0, _FPAD // _CH, floop, acc)
    acc = lax.fori_loop(0, _BPAD // _CH, bloop, acc)
    out_ref[:, :] = jnp.sum(acc, axis=1, keepdims=True)


@functools.partial(jax.jit, static_argnames=())
def kernel(fluidPositions, boundaryPositions, fluidFeatures, boundaryFeatures,
           W_fluid, W_boundary, support):
    f32 = jnp.float32
    big = jnp.float32(1e9)

    def pad_row(x, n, val):
        return jnp.pad(x, (0, n - x.shape[0]), constant_values=val).reshape(1, n)

    tx = jnp.pad(fluidPositions[:, 0], (0, _FPAD - _NF)).reshape(_FPAD, 1)
    ty = jnp.pad(fluidPositions[:, 1], (0, _FPAD - _NF)).reshape(_FPAD, 1)
    fsx = pad_row(fluidPositions[:, 0], _FPAD, big)
    fsy = pad_row(fluidPositions[:, 1], _FPAD, big)
    fsf = pad_row(fluidFeatures[:, 0], _FPAD, 0.0)
    bsx = pad_row(boundaryPositions[:, 0], _BPAD, big)
    bsy = pad_row(boundaryPositions[:, 1], _BPAD, big)
    bsf = pad_row(boundaryFeatures[:, 0], _BPAD, 0.0)
    sup = jnp.asarray(support, f32).reshape(1)
    wf = W_fluid.reshape(_RBF, _RBF).astype(f32)
    wb = W_boundary.reshape(_RBF, _RBF).astype(f32)

    grid = (_FPAD // _TT,)
    smem = pl.BlockSpec(memory_space=pltpu.SMEM)
    full_f = pl.BlockSpec((1, _FPAD), lambda i: (0, 0))
    full_b = pl.BlockSpec((1, _BPAD), lambda i: (0, 0))
    tgt = pl.BlockSpec((_TT, 1), lambda i: (i, 0))

    out = pl.pallas_call(
        _dense_kernel,
        grid=grid,
        in_specs=[smem, smem, smem, tgt, tgt,
                  full_f, full_f, full_f, full_b, full_b, full_b],
        out_specs=pl.BlockSpec((_TT, 1), lambda i: (i, 0)),
        out_shape=jax.ShapeDtypeStruct((_FPAD, 1), f32),
        compiler_params=pltpu.CompilerParams(
            dimension_semantics=("parallel",),
        ),
    )(sup, wf, wb, tx, ty, fsx, fsy, fsf, bsx, bsy, bsf)
    return out[:_NF]


# speedup vs baseline: 8.4073x; 8.4073x over previous
import jax
import jax.numpy as jnp
import numpy as np
from jax import lax
from jax.experimental import pallas as pl
from jax.experimental.pallas import tpu as pltpu

_TT = 256
_CH = 256
_NF = 10000
_NB = 2000
_FPAD = 10240
_BPAD = 2048
_RBF = 8


def _pair_acc(acc, tx, ty, sx, sy, sf, w, rsq, inv_s):
    dx = sx - tx
    dy = sy - ty
    d2 = dx * dx + dy * dy
    mask = (d2 < rsq).astype(jnp.float32)
    r = jnp.sqrt(d2) * inv_s
    u = 2.0 * r - 1.0
    v = jnp.arctan2(dy, dx) * jnp.float32(1.0 / np.pi)
    h_inv = jnp.float32((_RBF - 1) / 2.0)
    centers = np.linspace(-1.0, 1.0, _RBF)
    bv = [jnp.maximum(0.0, 1.0 - jnp.abs(v - jnp.float32(c)) * h_inv)
          for c in centers]
    t = jnp.zeros_like(d2)
    for n in range(_RBF):
        bu = jnp.maximum(0.0, 1.0 - jnp.abs(u - jnp.float32(centers[n])) * h_inv)
        row = jnp.zeros_like(d2)
        for m in range(_RBF):
            row = row + w[n][m] * bv[m]
        t = t + bu * row
    return acc + t * (mask * sf)


def _banded_kernel(sup_ref, wf_ref, wb_ref, tx_ref, ty_ref,
                   fsx_ref, fsy_ref, fsf_ref, bsx_ref, bsy_ref, bsf_ref,
                   out_ref):
    tx = tx_ref[:, :]
    ty = ty_ref[:, :]
    s = sup_ref[0]
    rsq = s * s
    inv_s = 1.0 / s
    wf = [[wf_ref[n, m] for m in range(_RBF)] for n in range(_RBF)]
    wb = [[wb_ref[n, m] for m in range(_RBF)] for n in range(_RBF)]

    lo = jnp.min(tx) - s
    hi = jnp.max(tx) + s

    def chunk_range(sx_row):
        start = jnp.sum((sx_row < lo).astype(jnp.int32))
        end = jnp.sum((sx_row < hi).astype(jnp.int32))
        k0 = start // _CH
        k1 = (end + _CH - 1) // _CH
        return k0, k1

    fk0, fk1 = chunk_range(fsx_ref[:, :])
    bk0, bk1 = chunk_range(bsx_ref[:, :])

    def floop(k, acc):
        sx = fsx_ref[:, pl.ds(k * _CH, _CH)]
        sy = fsy_ref[:, pl.ds(k * _CH, _CH)]
        sf = fsf_ref[:, pl.ds(k * _CH, _CH)]
        return _pair_acc(acc, tx, ty, sx, sy, sf, wf, rsq, inv_s)

    def bloop(k, acc):
        sx = bsx_ref[:, pl.ds(k * _CH, _CH)]
        sy = bsy_ref[:, pl.ds(k * _CH, _CH)]
        sf = bsf_ref[:, pl.ds(k * _CH, _CH)]
        return _pair_acc(acc, tx, ty, sx, sy, sf, wb, rsq, inv_s)

    acc = jnp.zeros((_TT, _CH), jnp.float32)
    acc = lax.fori_loop(fk0, fk1, floop, acc)
    acc = lax.fori_loop(bk0, bk1, bloop, acc)
    out_ref[:, :] = jnp.sum(acc, axis=1, keepdims=True)


def kernel(fluidPositions, boundaryPositions, fluidFeatures, boundaryFeatures,
           W_fluid, W_boundary, support):
    f32 = jnp.float32

    perm_f = jnp.argsort(fluidPositions[:, 0])
    fp = fluidPositions[perm_f]
    ff = fluidFeatures[perm_f]
    perm_b = jnp.argsort(boundaryPositions[:, 0])
    bp = boundaryPositions[perm_b]
    bf = boundaryFeatures[perm_b]

    def pad_row(x, n, val):
        return jnp.pad(x, (0, n - x.shape[0]), constant_values=val).reshape(1, n)

    tx = jnp.pad(fp[:, 0], (0, _FPAD - _NF), constant_values=2.0).reshape(_FPAD, 1)
    ty = jnp.pad(fp[:, 1], (0, _FPAD - _NF)).reshape(_FPAD, 1)
    fsx = pad_row(fp[:, 0], _FPAD, 2.0)
    fsy = pad_row(fp[:, 1], _FPAD, 0.0)
    fsf = pad_row(ff[:, 0], _FPAD, 0.0)
    bsx = pad_row(bp[:, 0], _BPAD, 2.0)
    bsy = pad_row(bp[:, 1], _BPAD, 0.0)
    bsf = pad_row(bf[:, 0], _BPAD, 0.0)
    sup = jnp.asarray(support, f32).reshape(1)
    wf = W_fluid.reshape(_RBF, _RBF).astype(f32)
    wb = W_boundary.reshape(_RBF, _RBF).astype(f32)

    grid = (_FPAD // _TT,)
    smem = pl.BlockSpec(memory_space=pltpu.SMEM)
    full_f = pl.BlockSpec((1, _FPAD), lambda i: (0, 0))
    full_b = pl.BlockSpec((1, _BPAD), lambda i: (0, 0))
    tgt = pl.BlockSpec((_TT, 1), lambda i: (i, 0))

    out_sorted = pl.pallas_call(
        _banded_kernel,
        grid=grid,
        in_specs=[smem, smem, smem, tgt, tgt,
                  full_f, full_f, full_f, full_b, full_b, full_b],
        out_specs=pl.BlockSpec((_TT, 1), lambda i: (i, 0)),
        out_shape=jax.ShapeDtypeStruct((_FPAD, 1), f32),
        compiler_params=pltpu.CompilerParams(
            dimension_semantics=("arbitrary",),
        ),
    )(sup, wf, wb, tx, ty, fsx, fsy, fsf, bsx, bsy, bsf)

    return jnp.zeros((_NF, 1), f32).at[perm_f].set(out_sorted[:_NF])


# speedup vs baseline: 18.4102x; 2.1898x over previous
import jax
import jax.numpy as jnp
import numpy as np
from jax import lax
from jax.experimental import pallas as pl
from jax.experimental.pallas import tpu as pltpu

_TT = 256
_CH = 256
_NF = 10000
_NB = 2000
_FPAD = 10240
_BPAD = 2048
_RBF = 8


def _pair_acc(acc, tx, ty, sx, sy, sf, wflat, rsq, inv_s):
    dx = sx - tx
    dy = sy - ty
    d2 = dx * dx + dy * dy
    mask = (d2 < rsq).astype(jnp.float32)
    r = jnp.sqrt(d2) * inv_s
    u = 2.0 * r - 1.0
    v = jnp.arctan2(dy, dx) * jnp.float32(1.0 / np.pi)
    h_inv = jnp.float32((_RBF - 1) / 2.0)
    tu = jnp.clip((u + 1.0) * h_inv, 0.0, jnp.float32(_RBF - 1))
    tv = jnp.clip((v + 1.0) * h_inv, 0.0, jnp.float32(_RBF - 1))
    iu = jnp.minimum(tu.astype(jnp.int32), _RBF - 2)
    iv = jnp.minimum(tv.astype(jnp.int32), _RBF - 2)
    fu = tu - iu.astype(jnp.float32)
    fv = tv - iv.astype(jnp.float32)
    idx = iu * _RBF + iv
    w2d = jnp.broadcast_to(wflat.reshape(1, _RBF * _RBF), (idx.shape[0], _RBF * _RBF))

    def gat(i):
        return jnp.take_along_axis(w2d, i, axis=1, mode="promise_in_bounds")

    w00 = gat(idx)
    w01 = gat(idx + 1)
    w10 = gat(idx + _RBF)
    w11 = gat(idx + _RBF + 1)
    t = ((1.0 - fu) * ((1.0 - fv) * w00 + fv * w01)
         + fu * ((1.0 - fv) * w10 + fv * w11))
    return acc + t * (mask * sf)


def _banded_kernel(sup_ref, wf_ref, wb_ref, tx_ref, ty_ref,
                   fsx_ref, fsy_ref, fsf_ref, bsx_ref, bsy_ref, bsf_ref,
                   out_ref):
    tx = tx_ref[:, :]
    ty = ty_ref[:, :]
    s = sup_ref[0]
    rsq = s * s
    inv_s = 1.0 / s
    wf = wf_ref[:]
    wb = wb_ref[:]

    lo = jnp.min(tx) - s
    hi = jnp.max(tx) + s

    def chunk_range(sx_row):
        start = jnp.sum((sx_row < lo).astype(jnp.int32))
        end = jnp.sum((sx_row < hi).astype(jnp.int32))
        k0 = start // _CH
        k1 = (end + _CH - 1) // _CH
        return k0, k1

    fk0, fk1 = chunk_range(fsx_ref[:, :])
    bk0, bk1 = chunk_range(bsx_ref[:, :])

    def floop(k, acc):
        sx = fsx_ref[:, pl.ds(k * _CH, _CH)]
        sy = fsy_ref[:, pl.ds(k * _CH, _CH)]
        sf = fsf_ref[:, pl.ds(k * _CH, _CH)]
        return _pair_acc(acc, tx, ty, sx, sy, sf, wf, rsq, inv_s)

    def bloop(k, acc):
        sx = bsx_ref[:, pl.ds(k * _CH, _CH)]
        sy = bsy_ref[:, pl.ds(k * _CH, _CH)]
        sf = bsf_ref[:, pl.ds(k * _CH, _CH)]
        return _pair_acc(acc, tx, ty, sx, sy, sf, wb, rsq, inv_s)

    acc = jnp.zeros((_TT, _CH), jnp.float32)
    acc = lax.fori_loop(fk0, fk1, floop, acc)
    acc = lax.fori_loop(bk0, bk1, bloop, acc)
    out_ref[:, :] = jnp.sum(acc, axis=1, keepdims=True)


def kernel(fluidPositions, boundaryPositions, fluidFeatures, boundaryFeatures,
           W_fluid, W_boundary, support):
    f32 = jnp.float32

    perm_f = jnp.argsort(fluidPositions[:, 0])
    fp = fluidPositions[perm_f]
    ff = fluidFeatures[perm_f]
    perm_b = jnp.argsort(boundaryPositions[:, 0])
    bp = boundaryPositions[perm_b]
    bf = boundaryFeatures[perm_b]

    def pad_row(x, n, val):
        return jnp.pad(x, (0, n - x.shape[0]), constant_values=val).reshape(1, n)

    tx = jnp.pad(fp[:, 0], (0, _FPAD - _NF), constant_values=2.0).reshape(_FPAD, 1)
    ty = jnp.pad(fp[:, 1], (0, _FPAD - _NF)).reshape(_FPAD, 1)
    fsx = pad_row(fp[:, 0], _FPAD, 2.0)
    fsy = pad_row(fp[:, 1], _FPAD, 0.0)
    fsf = pad_row(ff[:, 0], _FPAD, 0.0)
    bsx = pad_row(bp[:, 0], _BPAD, 2.0)
    bsy = pad_row(bp[:, 1], _BPAD, 0.0)
    bsf = pad_row(bf[:, 0], _BPAD, 0.0)
    sup = jnp.asarray(support, f32).reshape(1)
    wf = W_fluid.reshape(_RBF * _RBF).astype(f32)
    wb = W_boundary.reshape(_RBF * _RBF).astype(f32)

    grid = (_FPAD // _TT,)
    smem = pl.BlockSpec(memory_space=pltpu.SMEM)
    wspec = pl.BlockSpec((_RBF * _RBF,), lambda i: (0,))
    full_f = pl.BlockSpec((1, _FPAD), lambda i: (0, 0))
    full_b = pl.BlockSpec((1, _BPAD), lambda i: (0, 0))
    tgt = pl.BlockSpec((_TT, 1), lambda i: (i, 0))

    out_sorted = pl.pallas_call(
        _banded_kernel,
        grid=grid,
        in_specs=[smem, wspec, wspec, tgt, tgt,
                  full_f, full_f, full_f, full_b, full_b, full_b],
        out_specs=pl.BlockSpec((_TT, 1), lambda i: (i, 0)),
        out_shape=jax.ShapeDtypeStruct((_FPAD, 1), f32),
        compiler_params=pltpu.CompilerParams(
            dimension_semantics=("arbitrary",),
        ),
    )(sup, wf, wb, tx, ty, fsx, fsy, fsf, bsx, bsy, bsf)

    return jnp.zeros((_NF, 1), f32).at[perm_f].set(out_sorted[:_NF])


# speedup vs baseline: 20.1451x; 1.0942x over previous
import jax
import jax.numpy as jnp
import numpy as np
from jax import lax
from jax.experimental import pallas as pl
from jax.experimental.pallas import tpu as pltpu

_TT = 256
_CH = 256
_NF = 10000
_NB = 2000
_FPAD = 10240
_BPAD = 2048
_RBF = 8


_ATAN_C = (0.9999772197188205, -0.3326228337800521, 0.19354039031965328,
           -0.1164264883950182, 0.05264734009558123, -0.011719126877656156)


def _atan2(dy, dx):
    ax = jnp.abs(dx)
    ay = jnp.abs(dy)
    hi = jnp.maximum(ax, ay)
    lo = jnp.minimum(ax, ay)
    a = lo / jnp.maximum(hi, jnp.float32(1e-30))
    s = a * a
    p = jnp.float32(_ATAN_C[5])
    for c in _ATAN_C[4::-1]:
        p = p * s + jnp.float32(c)
    p = p * a
    r = jnp.where(ay > ax, jnp.float32(np.pi / 2) - p, p)
    r = jnp.where(dx < 0.0, jnp.float32(np.pi) - r, r)
    return jnp.where(dy < 0.0, -r, r)


def _pair_acc(acc, tx, ty, sx, sy, sf, wflat, rsq, inv_s):
    dx = sx - tx
    dy = sy - ty
    d2 = dx * dx + dy * dy
    mask = (d2 < rsq).astype(jnp.float32)
    h_inv = jnp.float32((_RBF - 1) / 2.0)
    tu = jnp.minimum(jnp.sqrt(d2) * (2.0 * h_inv * inv_s), jnp.float32(_RBF - 1))
    v = _atan2(dy, dx) * jnp.float32(1.0 / np.pi)
    tv = jnp.clip((v + 1.0) * h_inv, 0.0, jnp.float32(_RBF - 1))
    iu = jnp.minimum(tu.astype(jnp.int32), _RBF - 2)
    iv = jnp.minimum(tv.astype(jnp.int32), _RBF - 2)
    fu = tu - iu.astype(jnp.float32)
    fv = tv - iv.astype(jnp.float32)
    idx = iu * _RBF + iv
    w2d = jnp.broadcast_to(wflat.reshape(1, _RBF * _RBF), (idx.shape[0], _RBF * _RBF))

    def gat(i):
        return jnp.take_along_axis(w2d, i, axis=1, mode="promise_in_bounds")

    w00 = gat(idx)
    w01 = gat(idx + 1)
    w10 = gat(idx + _RBF)
    w11 = gat(idx + _RBF + 1)
    t = ((1.0 - fu) * ((1.0 - fv) * w00 + fv * w01)
         + fu * ((1.0 - fv) * w10 + fv * w11))
    return acc + t * (mask * sf)


def _banded_kernel(sup_ref, wf_ref, wb_ref, tx_ref, ty_ref,
                   fsx_ref, fsy_ref, fsf_ref, bsx_ref, bsy_ref, bsf_ref,
                   out_ref):
    tx = tx_ref[:, :]
    ty = ty_ref[:, :]
    s = sup_ref[0]
    rsq = s * s
    inv_s = 1.0 / s
    wf = wf_ref[:]
    wb = wb_ref[:]

    lo = jnp.min(tx) - s
    hi = jnp.max(tx) + s

    def chunk_range(sx_row):
        start = jnp.sum((sx_row < lo).astype(jnp.int32))
        end = jnp.sum((sx_row < hi).astype(jnp.int32))
        k0 = start // _CH
        k1 = (end + _CH - 1) // _CH
        return k0, k1

    fk0, fk1 = chunk_range(fsx_ref[:, :])
    bk0, bk1 = chunk_range(bsx_ref[:, :])

    def floop(k, acc):
        sx = fsx_ref[:, pl.ds(k * _CH, _CH)]
        sy = fsy_ref[:, pl.ds(k * _CH, _CH)]
        sf = fsf_ref[:, pl.ds(k * _CH, _CH)]
        return _pair_acc(acc, tx, ty, sx, sy, sf, wf, rsq, inv_s)

    def bloop(k, acc):
        sx = bsx_ref[:, pl.ds(k * _CH, _CH)]
        sy = bsy_ref[:, pl.ds(k * _CH, _CH)]
        sf = bsf_ref[:, pl.ds(k * _CH, _CH)]
        return _pair_acc(acc, tx, ty, sx, sy, sf, wb, rsq, inv_s)

    acc = jnp.zeros((_TT, _CH), jnp.float32)
    acc = lax.fori_loop(fk0, fk1, floop, acc)
    acc = lax.fori_loop(bk0, bk1, bloop, acc)
    out_ref[:, :] = jnp.sum(acc, axis=1, keepdims=True)


def kernel(fluidPositions, boundaryPositions, fluidFeatures, boundaryFeatures,
           W_fluid, W_boundary, support):
    f32 = jnp.float32

    perm_f = jnp.argsort(fluidPositions[:, 0])
    fp = fluidPositions[perm_f]
    ff = fluidFeatures[perm_f]
    perm_b = jnp.argsort(boundaryPositions[:, 0])
    bp = boundaryPositions[perm_b]
    bf = boundaryFeatures[perm_b]

    def pad_row(x, n, val):
        return jnp.pad(x, (0, n - x.shape[0]), constant_values=val).reshape(1, n)

    tx = jnp.pad(fp[:, 0], (0, _FPAD - _NF), constant_values=2.0).reshape(_FPAD, 1)
    ty = jnp.pad(fp[:, 1], (0, _FPAD - _NF)).reshape(_FPAD, 1)
    fsx = pad_row(fp[:, 0], _FPAD, 2.0)
    fsy = pad_row(fp[:, 1], _FPAD, 0.0)
    fsf = pad_row(ff[:, 0], _FPAD, 0.0)
    bsx = pad_row(bp[:, 0], _BPAD, 2.0)
    bsy = pad_row(bp[:, 1], _BPAD, 0.0)
    bsf = pad_row(bf[:, 0], _BPAD, 0.0)
    sup = jnp.asarray(support, f32).reshape(1)
    wf = W_fluid.reshape(_RBF * _RBF).astype(f32)
    wb = W_boundary.reshape(_RBF * _RBF).astype(f32)

    grid = (_FPAD // _TT,)
    smem = pl.BlockSpec(memory_space=pltpu.SMEM)
    wspec = pl.BlockSpec((_RBF * _RBF,), lambda i: (0,))
    full_f = pl.BlockSpec((1, _FPAD), lambda i: (0, 0))
    full_b = pl.BlockSpec((1, _BPAD), lambda i: (0, 0))
    tgt = pl.BlockSpec((_TT, 1), lambda i: (i, 0))

    out_sorted = pl.pallas_call(
        _banded_kernel,
        grid=grid,
        in_specs=[smem, wspec, wspec, tgt, tgt,
                  full_f, full_f, full_f, full_b, full_b, full_b],
        out_specs=pl.BlockSpec((_TT, 1), lambda i: (i, 0)),
        out_shape=jax.ShapeDtypeStruct((_FPAD, 1), f32),
        compiler_params=pltpu.CompilerParams(
            dimension_semantics=("arbitrary",),
        ),
    )(sup, wf, wb, tx, ty, fsx, fsy, fsf, bsx, bsy, bsf)

    return jnp.zeros((_NF, 1), f32).at[perm_f].set(out_sorted[:_NF])
